# baseline (speedup 1.0000x reference)
import functools

import jax
import jax.numpy as jnp
from jax import lax
from jax.experimental import pallas as pl
from jax.experimental.pallas import tpu as pltpu
from jax.experimental.pallas import tpu_sc as plsc

VOCAB = 100000
HIDDEN = 64
N_TOK = 512

_NC, _NS = 2, 16
_NW = _NC * _NS
_B_PER_W = N_TOK // _NW

VBLK = 4096


@functools.cache
def _make_sc_gather():
    mesh = plsc.VectorSubcoreMesh(core_axis_name="c", subcore_axis_name="s")

    @functools.partial(
        pl.kernel,
        mesh=mesh,
        out_type=jax.ShapeDtypeStruct((N_TOK, 128), jnp.int32),
        scratch_types=[
            pltpu.VMEM((_B_PER_W,), jnp.int32),
            pltpu.VMEM((_B_PER_W, 128), jnp.int32),
            pltpu.SemaphoreType.DMA,
        ],
    )
    def gather_kernel(table_hbm, idx_hbm, out_hbm, idx_v, rows_v, sem):
        wid = lax.axis_index("s") * _NC + lax.axis_index("c")
        base = wid * _B_PER_W
        pltpu.sync_copy(idx_hbm.at[pl.ds(base, _B_PER_W)], idx_v)
        pltpu.async_copy(table_hbm.at[idx_v], rows_v, sem).wait()
        pltpu.sync_copy(rows_v, out_hbm.at[pl.ds(base, _B_PER_W)])

    return gather_kernel


def _head_kernel(h4_ref, m0_ref, m1_ref, wt_ref, b_ref, o_ref):
    m0 = m0_ref[...] > 0.5
    m1 = m1_ref[...] > 0.5
    h4 = h4_ref[...]
    hlo = jnp.where(m0, h4[:, HIDDEN:2 * HIDDEN], h4[:, :HIDDEN])
    hhi = jnp.where(m0, h4[:, 3 * HIDDEN:], h4[:, 2 * HIDDEN:3 * HIDDEN])
    h = jnp.where(m1, hhi, hlo)
    o_ref[...] = lax.dot_general(
        h,
        wt_ref[...].astype(jnp.bfloat16),
        (((1,), (0,)), ((), ())),
        preferred_element_type=jnp.float32,
    ) + b_ref[...]


def kernel(input_ids, attention_mask, emb_table, W_head, b_head):
    del attention_mask
    ids = input_ids.reshape(N_TOK).astype(jnp.int32)
    ids_hi = ids // 4
    m0 = (ids % 2).astype(jnp.float32).reshape(N_TOK, 1)
    m1 = ((ids // 2) % 2).astype(jnp.float32).reshape(N_TOK, 1)

    tb = emb_table.astype(jnp.bfloat16)
    t32 = lax.bitcast_convert_type(tb.reshape(VOCAB, HIDDEN // 2, 2), jnp.int32)
    g = t32.reshape(VOCAB // 4, 128)

    g512 = _make_sc_gather()(g, ids_hi)
    h4 = lax.bitcast_convert_type(g512, jnp.bfloat16).reshape(N_TOK, 4 * HIDDEN)

    wt = W_head.T
    b2 = b_head.reshape(1, VOCAB)
    grid = (pl.cdiv(VOCAB, VBLK),)
    logits = pl.pallas_call(
        _head_kernel,
        grid=grid,
        in_specs=[
            pl.BlockSpec((N_TOK, 4 * HIDDEN), lambda j: (0, 0)),
            pl.BlockSpec((N_TOK, 1), lambda j: (0, 0)),
            pl.BlockSpec((N_TOK, 1), lambda j: (0, 0)),
            pl.BlockSpec((HIDDEN, VBLK), lambda j: (0, j)),
            pl.BlockSpec((1, VBLK), lambda j: (0, j)),
        ],
        out_specs=pl.BlockSpec((N_TOK, VBLK), lambda j: (0, j)),
        out_shape=jax.ShapeDtypeStruct((N_TOK, VOCAB), jnp.float32),
    )(h4, m0, m1, wt, b2)

    return logits.reshape(input_ids.shape[0], input_ids.shape[1], VOCAB)


# speedup vs baseline: 2.3883x; 2.3883x over previous
import functools

import jax
import jax.numpy as jnp
from jax import lax
from jax.experimental import pallas as pl
from jax.experimental.pallas import tpu as pltpu
from jax.experimental.pallas import tpu_sc as plsc

VOCAB = 100000
HIDDEN = 64
N_TOK = 512

_NC, _NS = 2, 16
_NW = _NC * _NS
_B_PER_W = N_TOK // _NW

VBLK = 4096


@functools.cache
def _make_sc_gather():
    mesh = plsc.VectorSubcoreMesh(core_axis_name="c", subcore_axis_name="s")

    @functools.partial(
        pl.kernel,
        mesh=mesh,
        out_type=jax.ShapeDtypeStruct((N_TOK, 2 * HIDDEN), jnp.float32),
        scratch_types=[
            pltpu.VMEM((_B_PER_W,), jnp.int32),
            pltpu.VMEM((_B_PER_W, 2 * HIDDEN), jnp.float32),
            pltpu.SemaphoreType.DMA,
        ],
    )
    def gather_kernel(table_hbm, idx_hbm, out_hbm, idx_v, rows_v, sem):
        wid = lax.axis_index("s") * _NC + lax.axis_index("c")
        base = wid * _B_PER_W
        pltpu.sync_copy(idx_hbm.at[pl.ds(base, _B_PER_W)], idx_v)
        pltpu.async_copy(table_hbm.at[idx_v], rows_v, sem).wait()
        pltpu.sync_copy(rows_v, out_hbm.at[pl.ds(base, _B_PER_W)])

    return gather_kernel


def _head_kernel(h2_ref, wt_ref, b_ref, o_ref):
    h = h2_ref[:, :HIDDEN].astype(jnp.bfloat16)
    o_ref[...] = lax.dot_general(
        h,
        wt_ref[...].astype(jnp.bfloat16),
        (((1,), (0,)), ((), ())),
        preferred_element_type=jnp.float32,
    ) + b_ref[...]


def kernel(input_ids, attention_mask, emb_table, W_head, b_head):
    del attention_mask
    ids = input_ids.reshape(N_TOK).astype(jnp.int32)

    tp = jnp.pad(emb_table, ((0, 0), (0, 2 * HIDDEN - HIDDEN)))

    h2 = _make_sc_gather()(tp, ids)

    wt = W_head.T
    b2 = b_head.reshape(1, VOCAB)
    grid = (pl.cdiv(VOCAB, VBLK),)
    logits = pl.pallas_call(
        _head_kernel,
        grid=grid,
        in_specs=[
            pl.BlockSpec((N_TOK, 2 * HIDDEN), lambda j: (0, 0)),
            pl.BlockSpec((HIDDEN, VBLK), lambda j: (0, j)),
            pl.BlockSpec((1, VBLK), lambda j: (0, j)),
        ],
        out_specs=pl.BlockSpec((N_TOK, VBLK), lambda j: (0, j)),
        out_shape=jax.ShapeDtypeStruct((N_TOK, VOCAB), jnp.float32),
    )(h2, wt, b2)

    return logits.reshape(input_ids.shape[0], input_ids.shape[1], VOCAB)
